# baseline (speedup 1.0000x reference)
import functools
import math

import jax
import jax.numpy as jnp
from jax import lax
from jax.experimental import pallas as pl
from jax.experimental.pallas import tpu as pltpu
from jax.experimental.pallas import tpu_sc as plsc

K = 8
R = 3
NEG = -1e38
NEG_GUARD = -1e37
LANES = 128


def _topk_body(BT, T, D, h_ref, idx_ref, sseg_ref):
    b = pl.program_id(0)
    i = pl.program_id(1)

    q = h_ref[0, pl.ds(i * BT, BT), :]
    rowloc = i * BT + lax.broadcasted_iota(jnp.int32, (BT, 1), 0)
    lane = lax.broadcasted_iota(jnp.int32, (BT, LANES), 1)
    colsl = lax.broadcasted_iota(jnp.int32, (BT, BT), 1)

    cv0 = jnp.full((BT, LANES), NEG, jnp.float32)
    ci0 = jnp.zeros((BT, LANES), jnp.int32)

    def seg_body(j, carry):
        cv, ci = carry
        kblk = h_ref[0, pl.ds(j * BT, BT), :]
        s = lax.dot_general(q, kblk, (((1,), (1,)), ((), ())),
                            preferred_element_type=jnp.float32)
        colsg = j * BT + colsl
        sseg_ref[...] = jnp.where(colsg <= rowloc, s, NEG)
        for kk in range(K):
            s = sseg_ref[...]
            m = jnp.max(s, axis=1, keepdims=True)
            pos = jnp.min(jnp.where(s == m, colsg, T), axis=1, keepdims=True)
            sseg_ref[...] = jnp.where(colsg == pos, NEG, s)
            c = j * K + kk
            cv = jnp.where(lane == c, m, cv)
            ci = jnp.where(lane == c, pos, ci)
        return cv, ci

    cv, ci = lax.fori_loop(0, i + 1, seg_body, (cv0, ci0))

    idx_acc = jnp.zeros((BT, LANES), jnp.int32)
    for kk in range(K):
        m = jnp.max(cv, axis=1, keepdims=True)
        pos = jnp.min(jnp.where(cv == m, lane, LANES), axis=1, keepdims=True)
        pick = lane == pos
        gi = jnp.sum(jnp.where(pick, ci, 0), axis=1, keepdims=True)
        gsel = jnp.where(m > NEG_GUARD, gi, rowloc) + b * T
        idx_acc = idx_acc + jnp.where(lane == kk, gsel, 0)
        cv = jnp.where(pick, NEG, cv)
    idx_ref[0] = idx_acc


def _topk_call(h, BT=256):
    B, T, D = h.shape
    NI = T // BT
    body = functools.partial(_topk_body, BT, T, D)
    return pl.pallas_call(
        body,
        grid=(B, NI),
        in_specs=[pl.BlockSpec((1, T, D), lambda b, i: (b, 0, 0))],
        out_specs=pl.BlockSpec((1, BT, LANES), lambda b, i: (b, i, 0)),
        out_shape=jax.ShapeDtypeStruct((B, T, LANES), jnp.int32),
        scratch_shapes=[pltpu.VMEM((BT, BT), jnp.float32)],
    )(h)


NC, NS = 2, 16
NW = NC * NS
CHUNK = 8


def _sc_gather_sum(h_flat, idx_flat):
    N, D = h_flat.shape
    per_w = N // NW
    n_chunks = per_w // CHUNK

    mesh = plsc.VectorSubcoreMesh(core_axis_name="c", subcore_axis_name="s")

    @functools.partial(
        pl.kernel, mesh=mesh,
        out_type=jax.ShapeDtypeStruct((N, D), jnp.float32),
        scratch_types=[
            pltpu.VMEM((CHUNK * K,), jnp.int32),
            pltpu.VMEM((CHUNK * K,), jnp.int32),
            pltpu.VMEM((CHUNK * K, D), jnp.float32),
            pltpu.VMEM((CHUNK * K, D), jnp.float32),
            pltpu.VMEM((CHUNK, D), jnp.float32),
            pltpu.SemaphoreType.DMA,
            pltpu.SemaphoreType.DMA,
        ],
    )
    def k(h_hbm, idx_hbm, out_hbm,
          idx_v0, idx_v1, rows_v0, rows_v1, msg_v, sem0, sem1):
        wid = lax.axis_index("s") * NC + lax.axis_index("c")
        idx_bufs = (idx_v0, idx_v1)
        row_bufs = (rows_v0, rows_v1)
        sems = (sem0, sem1)

        def issue(c):
            ibuf = idx_bufs[c % 2]
            tok0 = wid * per_w + c * CHUNK
            pltpu.sync_copy(idx_hbm.at[pl.ds(tok0 * K, CHUNK * K)], ibuf)
            return pltpu.async_copy(h_hbm.at[ibuf], row_bufs[c % 2],
                                    sems[c % 2])

        cps = [issue(0), None]
        for c in range(n_chunks):
            if c + 1 < n_chunks:
                cps[(c + 1) % 2] = issue(c + 1)
            cps[c % 2].wait()
            rows_v = row_bufs[c % 2]

            def tok_body(t, carry2):
                base = t * K

                def d_body(d, carry3):
                    for u in range(4):
                        sl = pl.ds((d * 4 + u) * 16, 16)
                        acc = rows_v[base, sl]
                        for kk in range(1, K):
                            acc = acc + rows_v[base + kk, sl]
                        msg_v[t, sl] = acc
                    return carry3

                lax.fori_loop(0, D // 64, d_body, 0)
                return carry2

            lax.fori_loop(0, CHUNK, tok_body, 0)
            tok0 = wid * per_w + c * CHUNK
            pltpu.sync_copy(msg_v, out_hbm.at[pl.ds(tok0, CHUNK)])

    return k(h_flat, idx_flat)


def _epi_body(r, is_last, BT, T, D,
              params_ref, h_ref, ms_ref, x_ref, gain_ref, bias_ref, out_ref):
    i = pl.program_id(1)
    mix = params_ref[r]
    momentum = params_ref[R]
    scale = params_ref[R + 1]

    h = h_ref[0]
    row1 = i * BT + lax.broadcasted_iota(jnp.int32, (BT, 1), 0)
    deg = jnp.minimum(row1.astype(jnp.float32) + 1.0, float(K))
    msg = (ms_ref[0] - (float(K) - deg) * h) / deg
    blended = mix * h + (1.0 - mix) * msg
    gb = blended * gain_ref[0] + bias_ref[0]
    act = gb * 0.5 * (1.0 + lax.erf(gb * (1.0 / math.sqrt(2.0))))
    hn = momentum * h + (1.0 - momentum) * act
    if is_last:
        out_ref[0] = (hn - x_ref[0]) * scale
    else:
        out_ref[0] = hn


def _epi_call(r, is_last, h, msgsum, x, gain_r, bias_r, params, BT=512):
    B, T, D = h.shape
    NI = T // BT
    body = functools.partial(_epi_body, r, is_last, BT, T, D)
    blk = pl.BlockSpec((1, BT, D), lambda b, i: (b, i, 0))
    return pl.pallas_call(
        body,
        grid=(B, NI),
        in_specs=[
            pl.BlockSpec(memory_space=pltpu.SMEM),
            blk, blk, blk,
            pl.BlockSpec((1, D), lambda b, i: (0, 0)),
            pl.BlockSpec((1, D), lambda b, i: (0, 0)),
        ],
        out_specs=blk,
        out_shape=jax.ShapeDtypeStruct((B, T, D), jnp.float32),
    )(params, h, msgsum, x, gain_r, bias_r)


def kernel(x, gain, bias, log_mix, log_momentum, log_scale):
    B, T, D = x.shape
    momentum = jax.nn.sigmoid(log_momentum)
    scale = jax.nn.softplus(log_scale) + 0.01
    mix = jax.nn.sigmoid(log_mix)
    params = jnp.concatenate(
        [mix.astype(jnp.float32),
         jnp.stack([momentum, scale]).astype(jnp.float32)])
    hs = [x[b:b + 1] for b in range(B)]
    xs = [x[b:b + 1] for b in range(B)]
    for r in range(R):
        gain_r = gain[r].reshape(1, D)
        bias_r = bias[r].reshape(1, D)
        idxs = [_topk_call(hs[b]) for b in range(B)]
        msgs = [
            _sc_gather_sum(hs[b].reshape(T, D),
                           idxs[b][:, :, :K].reshape(-1)).reshape(1, T, D)
            for b in range(B)
        ]
        hs = [_epi_call(r, r == R - 1, hs[b], msgs[b], xs[b],
                        gain_r, bias_r, params) for b in range(B)]
    return jnp.concatenate(hs, axis=0)


# speedup vs baseline: 2.8466x; 2.8466x over previous
import functools
import math

import jax
import jax.numpy as jnp
from jax.experimental import pallas as pl
from jax.experimental.pallas import tpu as pltpu

K = 8
R = 3
NEG = -1e38


def _round_body(r, is_last, BT, T, D, NI,
                params_ref, h_ref, *rest):
    if is_last:
        x_ref, gain_ref, bias_ref, out_ref, s_ref, m_ref = rest
    else:
        gain_ref, bias_ref, out_ref, s_ref, m_ref = rest
    i = pl.program_id(1)
    mix = params_ref[r]
    momentum = params_ref[R]
    scale = params_ref[R + 1]

    q = h_ref[0, pl.ds(i * BT, BT), :]

    def fill(j, carry):
        kblk = h_ref[0, pl.ds(j * BT, BT), :]
        s_ref[:, pl.ds(j * BT, BT)] = jax.lax.dot_general(
            q, kblk, (((1,), (1,)), ((), ())),
            preferred_element_type=jnp.float32)
        return carry

    jax.lax.fori_loop(0, i + 1, fill, 0)

    rows = i * BT + jax.lax.broadcasted_iota(jnp.int32, (BT, T), 0)
    cols = jax.lax.broadcasted_iota(jnp.int32, (BT, T), 1)
    causal = cols <= rows
    s_ref[...] = jnp.where(causal, s_ref[...], NEG)

    for _ in range(K):
        s = s_ref[...]
        m = jnp.max(s, axis=1, keepdims=True)
        s_ref[...] = jnp.where(s == m, NEG, s)

    s_ref[...] = jnp.where(causal & (s_ref[...] == NEG), 1.0, 0.0)

    m_ref[...] = jnp.zeros((BT, D), jnp.float32)

    def agg(j, carry):
        ablk = s_ref[:, pl.ds(j * BT, BT)]
        hblk = h_ref[0, pl.ds(j * BT, BT), :]
        m_ref[...] += jax.lax.dot_general(
            ablk, hblk, (((1,), (0,)), ((), ())),
            preferred_element_type=jnp.float32)
        return carry

    jax.lax.fori_loop(0, i + 1, agg, 0)

    row1 = i * BT + jax.lax.broadcasted_iota(jnp.int32, (BT, 1), 0)
    deg = jnp.minimum(row1.astype(jnp.float32) + 1.0, float(K))
    msg = m_ref[...] / deg

    blended = mix * q + (1.0 - mix) * msg
    gb = blended * gain_ref[0] + bias_ref[0]
    act = gb * 0.5 * (1.0 + jax.lax.erf(gb * (1.0 / math.sqrt(2.0))))
    hn = momentum * q + (1.0 - momentum) * act
    if is_last:
        out_ref[0] = (hn - x_ref[0, pl.ds(i * BT, BT), :]) * scale
    else:
        out_ref[0] = hn


def _round_call(r, is_last, h, x, gain_r, bias_r, params, BT=256):
    B, T, D = h.shape
    NI = T // BT
    body = functools.partial(_round_body, r, is_last, BT, T, D, NI)
    in_specs = [
        pl.BlockSpec(memory_space=pltpu.SMEM),
        pl.BlockSpec((1, T, D), lambda b, i: (b, 0, 0)),
    ]
    args = [params, h]
    if is_last:
        in_specs.append(pl.BlockSpec((1, T, D), lambda b, i: (b, 0, 0)))
        args.append(x)
    in_specs += [
        pl.BlockSpec((1, D), lambda b, i: (0, 0)),
        pl.BlockSpec((1, D), lambda b, i: (0, 0)),
    ]
    args += [gain_r, bias_r]
    return pl.pallas_call(
        body,
        grid=(B, NI),
        in_specs=in_specs,
        out_specs=pl.BlockSpec((1, BT, D), lambda b, i: (b, i, 0)),
        out_shape=jax.ShapeDtypeStruct((B, T, D), jnp.float32),
        scratch_shapes=[
            pltpu.VMEM((BT, T), jnp.float32),
            pltpu.VMEM((BT, D), jnp.float32),
        ],
    )(*args)


def kernel(x, gain, bias, log_mix, log_momentum, log_scale):
    B, T, D = x.shape
    momentum = jax.nn.sigmoid(log_momentum)
    scale = jax.nn.softplus(log_scale) + 0.01
    mix = jax.nn.sigmoid(log_mix)
    params = jnp.concatenate(
        [mix.astype(jnp.float32),
         jnp.stack([momentum, scale]).astype(jnp.float32)])
    h = x
    for r in range(R):
        h = _round_call(r, r == R - 1, h, x,
                        gain[r].reshape(1, D), bias[r].reshape(1, D), params)
    return h
